# Initial kernel scaffold; baseline (speedup 1.0000x reference)
import jax
import jax.numpy as jnp
from jax.experimental import pallas as pl


def kernel(nodes, to_neighs, u2e_weight, ge_w1, ge_b1, ge_w2, ge_b2):
    raise NotImplementedError("write your pallas kernel here")


# speedup vs baseline: 4.2076x; 4.2076x over previous
import functools

import jax
import jax.numpy as jnp
from jax import lax
from jax.experimental import pallas as pl
from jax.experimental.pallas import tpu as pltpu
from jax.experimental.pallas import tpu_sc as plsc


def _make_sc_gather(V, N, D):
    info = plsc.get_sparse_core_info()
    NC, NS = info.num_cores, info.num_subcores
    NW = NC * NS
    assert N % NW == 0
    n_per_w = N // NW
    CH = 128
    assert n_per_w % CH == 0
    n_ch = n_per_w // CH
    mesh = plsc.VectorSubcoreMesh(core_axis_name="c", subcore_axis_name="s")

    @functools.partial(
        pl.kernel,
        mesh=mesh,
        out_type=jax.ShapeDtypeStruct((N, D), jnp.float32),
        compiler_params=pltpu.CompilerParams(use_tc_tiling_on_sc=False),
        scratch_types=[
            pltpu.VMEM((n_per_w,), jnp.int32),
            pltpu.VMEM((CH, D), jnp.float32),
            pltpu.SemaphoreType.DMA,
        ],
    )
    def gather_k(idx_hbm, table_hbm, out_hbm, idx_v, rows_v, sem):
        wid = lax.axis_index("s") * NC + lax.axis_index("c")
        base = wid * n_per_w
        pltpu.sync_copy(idx_hbm.at[pl.ds(base, n_per_w)], idx_v)

        def body(c, carry):
            off = c * CH
            pltpu.async_copy(
                table_hbm.at[idx_v.at[pl.ds(off, CH)]], rows_v, sem
            ).wait()
            pltpu.sync_copy(rows_v, out_hbm.at[pl.ds(base + off, CH)])
            return carry

        lax.fori_loop(0, n_ch, body, 0)

    return gather_k


def _attn_body(m_ref, wbd_ref, b1b_ref, w2bd_ref, o_ref, s_ref):
    G = m_ref.shape[0]
    n_pairs = s_ref.shape[1] // 2
    D = o_ref.shape[1]
    wbd = wbd_ref[...]
    b1b = b1b_ref[...]
    w2bd = w2bd_ref[...]
    for k in range(n_pairs):
        pair = m_ref[:, 2 * D * k : 2 * D * (k + 1)]
        h = jnp.maximum(
            jnp.dot(pair, wbd, preferred_element_type=jnp.float32) + b1b, 0.0
        )
        s_ref[:, 2 * k : 2 * k + 2] = jnp.dot(
            h, w2bd, preferred_element_type=jnp.float32
        )
    sc = s_ref[...]
    mx = jnp.max(sc, axis=1, keepdims=True)
    e = jnp.exp(sc - mx)
    att = e / jnp.sum(e, axis=1, keepdims=True)
    lane = lax.broadcasted_iota(jnp.int32, (G, 2 * D), 1)
    low = lane < D
    acc = jnp.zeros((G, 2 * D), jnp.float32)
    for k in range(n_pairs):
        pair = m_ref[:, 2 * D * k : 2 * D * (k + 1)]
        a2 = jnp.where(low, att[:, 2 * k : 2 * k + 1], att[:, 2 * k + 1 : 2 * k + 2])
        acc = acc + a2 * pair
    o_ref[...] = acc[:, :D] + acc[:, D:]


def kernel(nodes, to_neighs, u2e_weight, ge_w1, ge_b1, ge_w2, ge_b2):
    B, M = nodes.shape
    V, D = u2e_weight.shape
    H = ge_w1.shape[1]

    idx = nodes.reshape(-1).astype(jnp.int32)
    staged = _make_sc_gather(V, B * M, D)(idx, u2e_weight)
    m2 = staged.reshape(B, M * D)

    z = jnp.zeros_like(ge_w1)
    wbd = jnp.concatenate(
        [jnp.concatenate([ge_w1, z], axis=1), jnp.concatenate([z, ge_w1], axis=1)],
        axis=0,
    )
    b1b = jnp.concatenate([ge_b1, ge_b1]).reshape(1, 2 * H)
    w2c = ge_w2[:, 0]
    w2bd = jnp.zeros((2 * H, 2), jnp.float32)
    w2bd = w2bd.at[:H, 0].set(w2c).at[H:, 1].set(w2c)

    Gt = 512
    grid = (B // Gt,)
    out = pl.pallas_call(
        _attn_body,
        grid=grid,
        in_specs=[
            pl.BlockSpec((Gt, M * D), lambda i: (i, 0)),
            pl.BlockSpec((2 * D, 2 * H), lambda i: (0, 0)),
            pl.BlockSpec((1, 2 * H), lambda i: (0, 0)),
            pl.BlockSpec((2 * H, 2), lambda i: (0, 0)),
        ],
        out_specs=pl.BlockSpec((Gt, D), lambda i: (i, 0)),
        out_shape=jax.ShapeDtypeStruct((B, D), jnp.float32),
        scratch_shapes=[pltpu.VMEM((Gt, M), jnp.float32)],
    )(m2, wbd, b1b, w2bd)
    return out


# speedup vs baseline: 5.3023x; 1.2602x over previous
import functools

import jax
import jax.numpy as jnp
from jax import lax
from jax.experimental import pallas as pl
from jax.experimental.pallas import tpu as pltpu
from jax.experimental.pallas import tpu_sc as plsc


def _make_sc_gather(V, M, B):
    info = plsc.get_sparse_core_info()
    NC, NS = info.num_cores, info.num_subcores
    NW = NC * NS
    N = M * B
    assert N % NW == 0
    n_per_w = N // NW
    CH = 128
    assert n_per_w % CH == 0 and B % CH == 0
    n_ch = n_per_w // CH
    mesh = plsc.VectorSubcoreMesh(core_axis_name="c", subcore_axis_name="s")

    @functools.partial(
        pl.kernel,
        mesh=mesh,
        out_type=jax.ShapeDtypeStruct((M, B, 128), jnp.float32),
        scratch_types=[
            pltpu.VMEM((n_per_w,), jnp.int32),
            pltpu.VMEM((2, CH, 128), jnp.float32),
            pltpu.SemaphoreType.DMA,
        ],
    )
    def gather_k(idx_hbm, table_hbm, out_hbm, idx_v, rows_v, sem):
        wid = lax.axis_index("s") * NC + lax.axis_index("c")
        base = wid * n_per_w
        pltpu.sync_copy(idx_hbm.at[pl.ds(base, n_per_w)], idx_v)

        def start(c):
            pltpu.async_copy(
                table_hbm.at[idx_v.at[pl.ds(c * CH, CH)]],
                rows_v.at[lax.rem(c, 2)],
                sem,
            )

        def drain_flush(c):
            buf = rows_v.at[lax.rem(c, 2)]
            pltpu.make_async_copy(
                table_hbm.at[idx_v.at[pl.ds(c * CH, CH)]], buf, sem
            ).wait()
            j = base + c * CH
            m_c = lax.div(j, B)
            g0 = lax.rem(j, B)
            pltpu.sync_copy(buf, out_hbm.at[m_c, pl.ds(g0, CH)])

        start(0)

        def body(c, carry):
            @pl.when(c + 1 < n_ch)
            def _():
                start(c + 1)

            drain_flush(c)
            return carry

        lax.fori_loop(0, n_ch, body, 0)

    return gather_k


def _attn_body(m_ref, w1_ref, b1_ref, w2t_ref, o_ref, s_ref):
    M = m_ref.shape[0]
    D = o_ref.shape[1]
    w1 = w1_ref[...]
    b1 = b1_ref[...]
    w2t = w2t_ref[...]
    for m in range(M):
        h = jnp.maximum(
            jnp.dot(m_ref[m], w1, preferred_element_type=jnp.float32) + b1, 0.0
        )
        s_ref[:, m : m + 1] = jnp.sum(h * w2t, axis=1, keepdims=True)
    sc = s_ref[...]
    mx = jnp.max(sc, axis=1, keepdims=True)
    e = jnp.exp(sc - mx)
    att = e / jnp.sum(e, axis=1, keepdims=True)
    acc = att[:, 0:1] * m_ref[0]
    for m in range(1, M):
        acc = acc + att[:, m : m + 1] * m_ref[m]
    o_ref[...] = acc[:, :D]


def kernel(nodes, to_neighs, u2e_weight, ge_w1, ge_b1, ge_w2, ge_b2):
    B, M = nodes.shape
    V, D = u2e_weight.shape
    H = ge_w1.shape[1]

    table128 = jnp.concatenate(
        [u2e_weight, jnp.zeros((V, 128 - D), jnp.float32)], axis=1
    )
    idx = nodes.T.reshape(-1).astype(jnp.int32)
    staged = _make_sc_gather(V, M, B)(idx, table128)

    b1r = ge_b1.reshape(1, H)
    w2t = ge_w2.reshape(1, H)
    w1p = jnp.concatenate(
        [ge_w1, jnp.zeros((128 - D, H), jnp.float32)], axis=0
    )

    Gt = 512
    grid = (B // Gt,)
    out = pl.pallas_call(
        _attn_body,
        grid=grid,
        in_specs=[
            pl.BlockSpec((M, Gt, 128), lambda i: (0, i, 0)),
            pl.BlockSpec((128, H), lambda i: (0, 0)),
            pl.BlockSpec((1, H), lambda i: (0, 0)),
            pl.BlockSpec((1, H), lambda i: (0, 0)),
        ],
        out_specs=pl.BlockSpec((Gt, D), lambda i: (i, 0)),
        out_shape=jax.ShapeDtypeStruct((B, D), jnp.float32),
        scratch_shapes=[pltpu.VMEM((Gt, M), jnp.float32)],
    )(staged, w1p, b1r, w2t)
    return out
